# baseline (speedup 1.0000x reference)
import functools
import math

import jax
import jax.numpy as jnp
from jax import lax
from jax.experimental import pallas as pl
from jax.experimental.pallas import tpu as pltpu
from jax.experimental.pallas import tpu_sc as plsc

_VOCAB = 1000000
_EMBED = 128
_MAX_LEN = 512
_B, _L = 1024, 200

_NC, _NS = 2, 16
_NW = _NC * _NS
_ROWS = _B * _L
_RPW = _ROWS // _NW
_CHUNK = 100
_NCHUNK = _RPW // _CHUNK


def _pos_table():
    position = jnp.arange(_MAX_LEN, dtype=jnp.float32)[:, None]
    div_term = jnp.exp(
        jnp.arange(0, _EMBED, 2, dtype=jnp.float32) * -(math.log(10000.0) / _EMBED)
    )
    pe = jnp.zeros((_MAX_LEN, _EMBED), dtype=jnp.float32)
    pe = pe.at[:, 0::2].set(jnp.sin(position * div_term))
    pe = pe.at[:, 1::2].set(jnp.cos(position * div_term))
    return pe[:_L]


_mesh = plsc.VectorSubcoreMesh(core_axis_name="c", subcore_axis_name="s")


@functools.partial(
    pl.kernel,
    out_type=jax.ShapeDtypeStruct((_ROWS, _EMBED), jnp.float32),
    mesh=_mesh,
    scratch_types=[
        pltpu.VMEM((_NCHUNK, _CHUNK), jnp.int32),
        pltpu.VMEM((_L, _EMBED), jnp.float32),
        pltpu.VMEM((_L, _EMBED), jnp.float32),
        pltpu.SemaphoreType.DMA,
    ],
)
def _embed_kernel(table_hbm, idx_hbm, pe_hbm, out_hbm, idx_v, pe_v, rows_v, sem):
    wid = lax.axis_index("s") * _NC + lax.axis_index("c")
    base = wid * _RPW
    pltpu.sync_copy(idx_hbm.at[wid], idx_v)
    pltpu.sync_copy(pe_hbm, pe_v)

    def pair_body(cc, carry):
        d0 = pltpu.async_copy(
            table_hbm.at[idx_v.at[2 * cc]], rows_v.at[pl.ds(0, _CHUNK)], sem)
        d1 = pltpu.async_copy(
            table_hbm.at[idx_v.at[2 * cc + 1]], rows_v.at[pl.ds(_CHUNK, _CHUNK)], sem)
        d0.wait()
        d1.wait()

        def row_body(i, carry2):
            for j in range(_EMBED // 16):
                cols = pl.ds(j * 16, 16)
                plsc.addupdate(rows_v.at[i, cols], pe_v[i, cols])
            return carry2

        lax.fori_loop(0, _L, row_body, 0, unroll=2)
        pltpu.sync_copy(rows_v, out_hbm.at[pl.ds(base + cc * _L, _L)])
        return carry

    lax.fori_loop(0, _NCHUNK // 2, pair_body, 0)


def kernel(sequence, token_table):
    idx = sequence.astype(jnp.int32).reshape(_NW, _NCHUNK, _CHUNK)
    pe = _pos_table()
    out = _embed_kernel(token_table, idx, pe)
    return out.reshape(_B, _L, _EMBED)


# speedup vs baseline: 1.4960x; 1.4960x over previous
import functools
import math

import jax
import jax.numpy as jnp
from jax import lax
from jax.experimental import pallas as pl
from jax.experimental.pallas import tpu as pltpu
from jax.experimental.pallas import tpu_sc as plsc

_VOCAB = 1000000
_EMBED = 128
_MAX_LEN = 512
_B, _L = 1024, 200

_NC, _NS = 2, 16
_NW = _NC * _NS
_ROWS = _B * _L
_RPW = _ROWS // _NW
_CHUNK = 100
_NCHUNK = _RPW // _CHUNK


def _pos_table():
    position = jnp.arange(_MAX_LEN, dtype=jnp.float32)[:, None]
    div_term = jnp.exp(
        jnp.arange(0, _EMBED, 2, dtype=jnp.float32) * -(math.log(10000.0) / _EMBED)
    )
    pe = jnp.zeros((_MAX_LEN, _EMBED), dtype=jnp.float32)
    pe = pe.at[:, 0::2].set(jnp.sin(position * div_term))
    pe = pe.at[:, 1::2].set(jnp.cos(position * div_term))
    return pe[:_L]


_mesh = plsc.VectorSubcoreMesh(core_axis_name="c", subcore_axis_name="s")


@functools.partial(
    pl.kernel,
    out_type=jax.ShapeDtypeStruct((_ROWS, _EMBED), jnp.float32),
    mesh=_mesh,
    scratch_types=[
        pltpu.VMEM((_NCHUNK, _CHUNK), jnp.int32),
        pltpu.VMEM((_L, _EMBED), jnp.float32),
        pltpu.VMEM((_L, _EMBED), jnp.float32),
        pltpu.VMEM((_L, _EMBED), jnp.float32),
        pltpu.SemaphoreType.DMA,
        pltpu.SemaphoreType.DMA,
        pltpu.SemaphoreType.DMA,
        pltpu.SemaphoreType.DMA,
    ],
)
def _embed_kernel(table_hbm, idx_hbm, pe_hbm, out_hbm,
                  idx_v, pe_v, rows_a, rows_b, sem_ia, sem_ib, sem_oa, sem_ob):
    wid = lax.axis_index("s") * _NC + lax.axis_index("c")
    base = wid * _RPW
    pltpu.sync_copy(idx_hbm.at[wid], idx_v)
    pltpu.sync_copy(pe_hbm, pe_v)

    def gather_pair(p, buf, sem):
        pltpu.async_copy(
            table_hbm.at[idx_v.at[2 * p]], buf.at[pl.ds(0, _CHUNK)], sem)
        pltpu.async_copy(
            table_hbm.at[idx_v.at[2 * p + 1]], buf.at[pl.ds(_CHUNK, _CHUNK)], sem)

    def wait_pair(buf, sem):
        pltpu.make_async_copy(
            table_hbm.at[idx_v.at[0]], buf.at[pl.ds(0, _CHUNK)], sem).wait()
        pltpu.make_async_copy(
            table_hbm.at[idx_v.at[0]], buf.at[pl.ds(_CHUNK, _CHUNK)], sem).wait()

    def issue_out(p, buf, sem):
        pltpu.async_copy(buf, out_hbm.at[pl.ds(base + p * _L, _L)], sem)

    def wait_out(buf, sem):
        pltpu.make_async_copy(buf, out_hbm.at[pl.ds(base, _L)], sem).wait()

    def add_pe(buf):
        def row_body(i, carry2):
            for j in range(_EMBED // 16):
                cols = pl.ds(j * 16, 16)
                plsc.addupdate(buf.at[i, cols], pe_v[i, cols])
            return carry2

        lax.fori_loop(0, _L, row_body, 0, unroll=2)

    npair = _NCHUNK // 2
    gather_pair(0, rows_a, sem_ia)

    def body(t, carry):
        pa = 2 * t
        @pl.when(t >= 1)
        def _():
            wait_out(rows_b, sem_ob)
        gather_pair(pa + 1, rows_b, sem_ib)
        wait_pair(rows_a, sem_ia)
        add_pe(rows_a)
        issue_out(pa, rows_a, sem_oa)
        @pl.when(t < npair // 2 - 1)
        def _():
            wait_out(rows_a, sem_oa)
            gather_pair(pa + 2, rows_a, sem_ia)
        wait_pair(rows_b, sem_ib)
        add_pe(rows_b)
        issue_out(pa + 1, rows_b, sem_ob)
        return carry

    lax.fori_loop(0, npair // 2, body, 0)
    wait_out(rows_a, sem_oa)
    wait_out(rows_b, sem_ob)


def kernel(sequence, token_table):
    idx = sequence.astype(jnp.int32).reshape(_NW, _NCHUNK, _CHUNK)
    pe = _pos_table()
    out = _embed_kernel(token_table, idx, pe)
    return out.reshape(_B, _L, _EMBED)


# speedup vs baseline: 1.7276x; 1.1548x over previous
import functools
import math

import jax
import jax.numpy as jnp
from jax import lax
from jax.experimental import pallas as pl
from jax.experimental.pallas import tpu as pltpu
from jax.experimental.pallas import tpu_sc as plsc

_VOCAB = 1000000
_EMBED = 128
_MAX_LEN = 512
_B, _L = 1024, 200

_NC, _NS = 2, 16
_NW = _NC * _NS
_ROWS = _B * _L
_RPW = _ROWS // _NW
_CHUNK = 100
_NCHUNK = _RPW // _CHUNK


def _pos_table():
    position = jnp.arange(_MAX_LEN, dtype=jnp.float32)[:, None]
    div_term = jnp.exp(
        jnp.arange(0, _EMBED, 2, dtype=jnp.float32) * -(math.log(10000.0) / _EMBED)
    )
    pe = jnp.zeros((_MAX_LEN, _EMBED), dtype=jnp.float32)
    pe = pe.at[:, 0::2].set(jnp.sin(position * div_term))
    pe = pe.at[:, 1::2].set(jnp.cos(position * div_term))
    return pe[:_L]


_mesh = plsc.VectorSubcoreMesh(core_axis_name="c", subcore_axis_name="s")


@functools.partial(
    pl.kernel,
    out_type=jax.ShapeDtypeStruct((_ROWS, _EMBED), jnp.float32),
    mesh=_mesh,
    scratch_types=[
        pltpu.VMEM((_NCHUNK, _CHUNK), jnp.int32),
        pltpu.VMEM((_L, _EMBED), jnp.float32),
        pltpu.VMEM((_L, _EMBED), jnp.float32),
        pltpu.VMEM((_L, _EMBED), jnp.float32),
        pltpu.VMEM((_L, _EMBED), jnp.float32),
        pltpu.SemaphoreType.DMA,
        pltpu.SemaphoreType.DMA,
        pltpu.SemaphoreType.DMA,
        pltpu.SemaphoreType.DMA,
        pltpu.SemaphoreType.DMA,
        pltpu.SemaphoreType.DMA,
    ],
)
def _embed_kernel(table_hbm, idx_hbm, pe_hbm, out_hbm,
                  idx_v, pe_v, rows_0, rows_1, rows_2,
                  sem_i0, sem_i1, sem_i2, sem_o0, sem_o1, sem_o2):
    wid = lax.axis_index("s") * _NC + lax.axis_index("c")
    base = wid * _RPW
    pltpu.sync_copy(idx_hbm.at[wid], idx_v)
    pltpu.sync_copy(pe_hbm, pe_v)

    bufs = (rows_0, rows_1, rows_2)
    sems_i = (sem_i0, sem_i1, sem_i2)
    sems_o = (sem_o0, sem_o1, sem_o2)
    _NBUF = 3

    def gather_pair(p, buf, sem):
        pltpu.async_copy(
            table_hbm.at[idx_v.at[2 * p]], buf.at[pl.ds(0, _CHUNK)], sem)
        pltpu.async_copy(
            table_hbm.at[idx_v.at[2 * p + 1]], buf.at[pl.ds(_CHUNK, _CHUNK)], sem)

    def wait_pair(buf, sem):
        pltpu.make_async_copy(
            table_hbm.at[idx_v.at[0]], buf.at[pl.ds(0, _CHUNK)], sem).wait()
        pltpu.make_async_copy(
            table_hbm.at[idx_v.at[0]], buf.at[pl.ds(_CHUNK, _CHUNK)], sem).wait()

    def issue_out(p, buf, sem):
        pltpu.async_copy(buf, out_hbm.at[pl.ds(base + p * _L, _L)], sem)

    def wait_out(buf, sem):
        pltpu.make_async_copy(buf, out_hbm.at[pl.ds(base, _L)], sem).wait()

    def add_pe(buf):
        def row_body(i, carry2):
            for j in range(_EMBED // 16):
                cols = pl.ds(j * 16, 16)
                plsc.addupdate(buf.at[i, cols], pe_v[i, cols])
            return carry2

        lax.fori_loop(0, _L, row_body, 0, unroll=2)

    def on_buf(sel, fn):
        for k in range(_NBUF):
            @pl.when(sel == k)
            def _(k=k):
                fn(bufs[k], sems_i[k], sems_o[k])

    npair = _NCHUNK // 2
    gather_pair(0, rows_0, sem_i0)
    gather_pair(1, rows_1, sem_i1)

    def body(t, carry):
        def process(buf, sem_i, sem_o):
            wait_pair(buf, sem_i)
            add_pe(buf)
            issue_out(t, buf, sem_o)

        on_buf(t % _NBUF, process)

        @pl.when(t + 2 < npair)
        def _():
            def prefetch(buf, sem_i, sem_o):
                @pl.when(t >= 1)
                def _():
                    wait_out(buf, sem_o)
                gather_pair(t + 2, buf, sem_i)

            on_buf((t + 2) % _NBUF, prefetch)

        return carry

    lax.fori_loop(0, npair, body, 0)
    for k in range(_NBUF):
        wait_out(bufs[k], sems_o[k])


def kernel(sequence, token_table):
    idx = sequence.astype(jnp.int32).reshape(_NW, _NCHUNK, _CHUNK)
    pe = _pos_table()
    out = _embed_kernel(token_table, idx, pe)
    return out.reshape(_B, _L, _EMBED)
